# baseline (speedup 1.0000x reference)
import functools

import jax
import jax.numpy as jnp
from jax.experimental import pallas as pl
from jax.experimental.pallas import tpu as pltpu

COMMITMENT_COST = 0.25

ROWS_PER_BLOCK = 1024


def _vq_block_kernel(x_ref, xt_ref, e_ref, et_ref, qst_ref, idx_ref,
                     loss_ref):
    x = x_ref[...]
    e = e_ref[...]
    et = et_ref[...]
    num_embeddings = e.shape[1]

    xsq = jnp.sum(x * x, axis=1, keepdims=True)
    esq = jnp.sum(e * e, axis=0, keepdims=True)
    prod_t = jax.lax.dot_general(
        et, xt_ref[...], dimension_numbers=(((1,), (0,)), ((), ())),
        preferred_element_type=jnp.float32)
    dist_t = xsq.T + esq.T - 2.0 * prod_t

    idx = jnp.argmin(dist_t, axis=0).astype(jnp.int32)
    idx_ref[...] = idx.reshape(idx_ref.shape)

    onehot_t = (jax.lax.broadcasted_iota(jnp.int32, dist_t.shape, 0)
                == idx[None, :]).astype(jnp.float32)
    quantized = jax.lax.dot_general(
        onehot_t, et, dimension_numbers=(((0,), (0,)), ((), ())),
        preferred_element_type=jnp.float32)

    qst_ref[...] = x + (quantized - x)

    diff = x - quantized
    loss_ref[...] = jnp.sum(diff * diff).reshape(1, 1, 1)


@functools.partial(jax.jit, static_argnames=())
def kernel(inputs, embeddings):
    embedding_dim = embeddings.shape[0]
    num_embeddings = embeddings.shape[1]
    flat = inputs.reshape(-1, embedding_dim)
    n_rows = flat.shape[0]
    n_blocks = n_rows // ROWS_PER_BLOCK

    embeddings_t = embeddings.T
    flat_t = flat.T

    grid = (n_blocks,)
    qst, idx2d, loss_sum = pl.pallas_call(
        _vq_block_kernel,
        grid=grid,
        in_specs=[
            pl.BlockSpec((ROWS_PER_BLOCK, embedding_dim), lambda i: (i, 0)),
            pl.BlockSpec((embedding_dim, ROWS_PER_BLOCK), lambda i: (0, i)),
            pl.BlockSpec((embedding_dim, num_embeddings), lambda i: (0, 0)),
            pl.BlockSpec((num_embeddings, embedding_dim), lambda i: (0, 0)),
        ],
        out_specs=[
            pl.BlockSpec((ROWS_PER_BLOCK, embedding_dim), lambda i: (i, 0)),
            pl.BlockSpec((1, 1, ROWS_PER_BLOCK), lambda i: (i, 0, 0)),
            pl.BlockSpec((1, 1, 1), lambda i: (i, 0, 0)),
        ],
        out_shape=[
            jax.ShapeDtypeStruct((n_rows, embedding_dim), jnp.float32),
            jax.ShapeDtypeStruct((n_blocks, 1, ROWS_PER_BLOCK), jnp.int32),
            jax.ShapeDtypeStruct((n_blocks, 1, 1), jnp.float32),
        ],
        compiler_params=pltpu.CompilerParams(
            dimension_semantics=("parallel",)),
    )(flat, flat_t, embeddings, embeddings_t)

    quantized_st = qst.reshape(inputs.shape)
    encoding_indices = idx2d.reshape(n_rows)
    mean_sq = jnp.sum(loss_sum) / jnp.float32(inputs.size)
    commitment_loss = COMMITMENT_COST * mean_sq
    codebook_loss = mean_sq
    return (quantized_st, encoding_indices, commitment_loss, codebook_loss)


# speedup vs baseline: 1.1067x; 1.1067x over previous
import functools

import jax
import jax.numpy as jnp
from jax.experimental import pallas as pl
from jax.experimental.pallas import tpu as pltpu

COMMITMENT_COST = 0.25

ROWS_PER_BLOCK = 1024


def _vq_block_kernel(x_ref, xt_ref, e_ref, et_ref, qst_ref, idx_ref,
                     loss_ref):
    x = x_ref[...]
    e = e_ref[...]
    et = et_ref[...]
    num_embeddings = e.shape[1]

    xt = xt_ref[...]
    xsq_t = jnp.sum(xt * xt, axis=0, keepdims=True)
    esq_t = jnp.sum(et * et, axis=1, keepdims=True)
    prod_t = jax.lax.dot_general(
        et, xt, dimension_numbers=(((1,), (0,)), ((), ())),
        preferred_element_type=jnp.float32)
    dist_t = xsq_t + esq_t - 2.0 * prod_t

    idx = jnp.argmin(dist_t, axis=0).astype(jnp.int32)
    idx_ref[...] = idx.reshape(idx_ref.shape)

    onehot_t = (jax.lax.broadcasted_iota(jnp.int32, dist_t.shape, 0)
                == idx[None, :]).astype(jnp.float32)
    quantized = jax.lax.dot_general(
        onehot_t, et, dimension_numbers=(((0,), (0,)), ((), ())),
        preferred_element_type=jnp.float32)

    qst_ref[...] = x + (quantized - x)

    diff = x - quantized
    loss_ref[...] = jnp.sum(diff * diff).reshape(1, 1, 1)


@functools.partial(jax.jit, static_argnames=())
def kernel(inputs, embeddings):
    embedding_dim = embeddings.shape[0]
    num_embeddings = embeddings.shape[1]
    flat = inputs.reshape(-1, embedding_dim)
    n_rows = flat.shape[0]
    n_blocks = n_rows // ROWS_PER_BLOCK

    embeddings_t = embeddings.T
    flat_t = flat.T

    grid = (n_blocks,)
    qst, idx2d, loss_sum = pl.pallas_call(
        _vq_block_kernel,
        grid=grid,
        in_specs=[
            pl.BlockSpec((ROWS_PER_BLOCK, embedding_dim), lambda i: (i, 0)),
            pl.BlockSpec((embedding_dim, ROWS_PER_BLOCK), lambda i: (0, i)),
            pl.BlockSpec((embedding_dim, num_embeddings), lambda i: (0, 0)),
            pl.BlockSpec((num_embeddings, embedding_dim), lambda i: (0, 0)),
        ],
        out_specs=[
            pl.BlockSpec((ROWS_PER_BLOCK, embedding_dim), lambda i: (i, 0)),
            pl.BlockSpec((1, 1, ROWS_PER_BLOCK), lambda i: (i, 0, 0)),
            pl.BlockSpec((1, 1, 1), lambda i: (i, 0, 0)),
        ],
        out_shape=[
            jax.ShapeDtypeStruct((n_rows, embedding_dim), jnp.float32),
            jax.ShapeDtypeStruct((n_blocks, 1, ROWS_PER_BLOCK), jnp.int32),
            jax.ShapeDtypeStruct((n_blocks, 1, 1), jnp.float32),
        ],
        compiler_params=pltpu.CompilerParams(
            dimension_semantics=("parallel",)),
    )(flat, flat_t, embeddings, embeddings_t)

    quantized_st = qst.reshape(inputs.shape)
    encoding_indices = idx2d.reshape(n_rows)
    mean_sq = jnp.sum(loss_sum) / jnp.float32(inputs.size)
    commitment_loss = COMMITMENT_COST * mean_sq
    codebook_loss = mean_sq
    return (quantized_st, encoding_indices, commitment_loss, codebook_loss)
